# baseline (speedup 1.0000x reference)
import functools

import jax
import jax.numpy as jnp
from jax import lax
from jax.experimental import pallas as pl
from jax.experimental.pallas import tpu as pltpu
from jax.experimental.pallas import tpu_sc as plsc

B, S, C = 2, 4096, 4096
NC, NS = 2, 16
NW = NC * NS
CPW = (B * C) // NW
L = 16
G = CPW // L
RCHUNK = 64
NCHUNK = S // RCHUNK
NBUF = 3


def _scan_body(x_hbm, o_hbm, bufs, in_sems, out_sems):
    cid = lax.axis_index("c")
    sid = lax.axis_index("s")
    wid = sid * NC + cid
    b = wid // NS
    col0 = (wid % NS) * CPW

    def in_copy(k):
        s = k % NBUF
        return pltpu.make_async_copy(
            x_hbm.at[b, pl.ds(k * RCHUNK, RCHUNK), pl.ds(col0, CPW)],
            bufs[s], in_sems[s])

    def out_copy(k):
        s = k % NBUF
        return pltpu.make_async_copy(
            bufs[s], o_hbm.at[b, pl.ds(k * RCHUNK, RCHUNK), pl.ds(col0, CPW)],
            out_sems[s])

    def compute(buf, carries):
        def row_body(r, carries):
            new = []
            for g in range(G):
                v = carries[g] + buf[r, pl.ds(g * L, L)]
                buf[r, pl.ds(g * L, L)] = v
                new.append(v)
            return tuple(new)
        return lax.fori_loop(0, RCHUNK, row_body, carries)

    in_copy(0).start()
    in_copy(1).start()
    carries = tuple(jnp.zeros((L,), jnp.float32) for _ in range(G))
    for k in range(NCHUNK):
        in_copy(k).wait()
        carries = compute(bufs[k % NBUF], carries)
        out_copy(k).start()
        nk = k + NBUF - 1
        if nk < NCHUNK:
            if nk - NBUF >= 0:
                out_copy(nk - NBUF).wait()
            in_copy(nk).start()
    for k in range(max(NCHUNK - NBUF, 0), NCHUNK):
        out_copy(k).wait()


@functools.partial(
    pl.kernel,
    out_type=jax.ShapeDtypeStruct((B, S, C), jnp.float32),
    mesh=plsc.VectorSubcoreMesh(core_axis_name="c", subcore_axis_name="s"),
    scratch_types=[
        [pltpu.VMEM((RCHUNK, CPW), jnp.float32) for _ in range(NBUF)],
        [pltpu.SemaphoreType.DMA for _ in range(NBUF)],
        [pltpu.SemaphoreType.DMA for _ in range(NBUF)],
    ],
)
def _cumsum_sc(x_hbm, o_hbm, bufs, in_sems, out_sems):
    _scan_body(x_hbm, o_hbm, bufs, in_sems, out_sems)


def kernel(x):
    return _cumsum_sc(x)


# speedup vs baseline: 1.0258x; 1.0258x over previous
import functools

import jax
import jax.numpy as jnp
from jax import lax
from jax.experimental import pallas as pl
from jax.experimental.pallas import tpu as pltpu
from jax.experimental.pallas import tpu_sc as plsc

B, S, C = 2, 4096, 4096
NC, NS = 2, 16
NW = NC * NS
CPW = (B * C) // NW
L = 16
G = CPW // L
RCHUNK = 168
NBUF = 3
_CHUNKS = []
_r = 0
while _r < S:
    _n = min(RCHUNK, S - _r)
    _CHUNKS.append((_r, _n))
    _r += _n
NCHUNK = len(_CHUNKS)


def _scan_body(x_hbm, o_hbm, bufs, in_sems, out_sems):
    cid = lax.axis_index("c")
    sid = lax.axis_index("s")
    wid = sid * NC + cid
    b = wid // NS
    col0 = (wid % NS) * CPW

    def in_copy(k):
        s = k % NBUF
        r0, n = _CHUNKS[k]
        return pltpu.make_async_copy(
            x_hbm.at[b, pl.ds(r0, n), pl.ds(col0, CPW)],
            bufs[s].at[pl.ds(0, n), :], in_sems[s])

    def out_copy(k):
        s = k % NBUF
        r0, n = _CHUNKS[k]
        return pltpu.make_async_copy(
            bufs[s].at[pl.ds(0, n), :],
            o_hbm.at[b, pl.ds(r0, n), pl.ds(col0, CPW)], out_sems[s])

    def compute(k, carries):
        buf = bufs[k % NBUF]
        _, n = _CHUNKS[k]

        def row_body(r, carries):
            new = []
            for g in range(G):
                v = carries[g] + buf[r, pl.ds(g * L, L)]
                buf[r, pl.ds(g * L, L)] = v
                new.append(v)
            return tuple(new)
        return lax.fori_loop(0, n, row_body, carries)

    in_copy(0).start()
    in_copy(1).start()
    carries = tuple(jnp.zeros((L,), jnp.float32) for _ in range(G))
    for k in range(NCHUNK):
        in_copy(k).wait()
        carries = compute(k, carries)
        out_copy(k).start()
        nk = k + NBUF - 1
        if nk < NCHUNK:
            if nk - NBUF >= 0:
                out_copy(nk - NBUF).wait()
            in_copy(nk).start()
    for k in range(max(NCHUNK - NBUF, 0), NCHUNK):
        out_copy(k).wait()


@functools.partial(
    pl.kernel,
    out_type=jax.ShapeDtypeStruct((B, S, C), jnp.float32),
    mesh=plsc.VectorSubcoreMesh(core_axis_name="c", subcore_axis_name="s"),
    scratch_types=[
        [pltpu.VMEM((RCHUNK, CPW), jnp.float32) for _ in range(NBUF)],
        [pltpu.SemaphoreType.DMA for _ in range(NBUF)],
        [pltpu.SemaphoreType.DMA for _ in range(NBUF)],
    ],
)
def _cumsum_sc(x_hbm, o_hbm, bufs, in_sems, out_sems):
    _scan_body(x_hbm, o_hbm, bufs, in_sems, out_sems)


def kernel(x):
    return _cumsum_sc(x)
